# baseline (speedup 1.0000x reference)
import jax
import jax.numpy as jnp
from jax.experimental import pallas as pl

_B, _N, _C = 256, 128, 10
_K = 10
_S = 8

_DN = (((1,), (0,)), ((), ()))


def _one_set(xb, xt, weights):
    (wr0, wo0, b0, wr1, wo1, b1, wr2, wo2, b2) = weights

    ones_row = jnp.ones((1, _N), jnp.float32)
    acc = jnp.zeros((_N, _N), jnp.float32)
    for c in range(_C):
        colb = jax.lax.dot_general(xb[:, c:c + 1], ones_row, _DN,
                                   preferred_element_type=jnp.float32,
                                   precision=jax.lax.Precision.HIGHEST)
        d = colb - xt[c:c + 1, :]
        acc = acc + d * d
    dist = jnp.sqrt(acc)

    iota = jax.lax.broadcasted_iota(jnp.int32, (_N, _N), 0).astype(jnp.float32)
    adj_t = jnp.zeros((_N, _N), jnp.float32)
    dcur = dist
    for t in range(_K + 1):
        v, ix = dcur, iota
        rows = _N
        while rows > 8:
            rows //= 2
            av, bv = v[:rows], v[rows:]
            ai, bi = ix[:rows], ix[rows:]
            take_b = bv < av
            v = jnp.minimum(av, bv)
            ix = jnp.where(take_b, bi, ai)
        for sh in (4, 2, 1):
            bv = jnp.roll(v, -sh, axis=0)
            bi = jnp.roll(ix, -sh, axis=0)
            take_b = (bv < v) | ((bv == v) & (bi < ix))
            v = jnp.where(take_b, bv, v)
            ix = jnp.where(take_b, bi, ix)
        onehot = iota == ix[0:1, :]
        if t > 0:
            adj_t = adj_t + jnp.where(onehot, 1.0, 0.0)
        dcur = jnp.where(onehot, jnp.inf, dcur)

    def conv(h, wr, wo, b):
        agg = jax.lax.dot_general(adj_t, h, _DN, preferred_element_type=jnp.float32)
        return (jax.lax.dot_general(agg, wr, _DN, preferred_element_type=jnp.float32)
                + b
                + jax.lax.dot_general(h, wo, _DN, preferred_element_type=jnp.float32))

    h = conv(xb, wr0, wo0, b0)
    h = jnp.maximum(h, 0.0)
    h = conv(h, wr1, wo1, b1)
    h = jnp.maximum(h, 0.0)
    h = conv(h, wr2, wo2, b2)

    return jax.lax.dot_general(h, h, (((1,), (1,)), ((), ())),
                               preferred_element_type=jnp.float32)


def _body(x_ref, xt_ref, wr0_ref, wo0_ref, b0_ref, wr1_ref, wo1_ref, b1_ref,
          wr2_ref, wo2_ref, b2_ref, out_ref):
    weights = (wr0_ref[...], wo0_ref[...], b0_ref[...],
               wr1_ref[...], wo1_ref[...], b1_ref[...],
               wr2_ref[...], wo2_ref[...], b2_ref[...])
    for s in range(_S):
        out_ref[s, 0] = _one_set(x_ref[s], xt_ref[s], weights)


def kernel(x, Wrel0, Wroot0, brel0, Wrel1, Wroot1, brel1, Wrel2, Wroot2, brel2):
    xt = jnp.transpose(x, (0, 2, 1))
    full = lambda s: pl.BlockSpec(s, lambda i: (0,) * len(s))
    grid_spec = pl.GridSpec(
        grid=(_B // _S,),
        in_specs=[
            pl.BlockSpec((_S, _N, _C), lambda i: (i, 0, 0)),
            pl.BlockSpec((_S, _C, _N), lambda i: (i, 0, 0)),
            full((10, 64)), full((10, 64)), full((1, 64)),
            full((64, 64)), full((64, 64)), full((1, 64)),
            full((64, 32)), full((64, 32)), full((1, 32)),
        ],
        out_specs=pl.BlockSpec((_S, 1, _N, _N), lambda i: (i, 0, 0, 0)),
    )
    out = pl.pallas_call(
        _body,
        grid_spec=grid_spec,
        out_shape=jax.ShapeDtypeStruct((_B, 1, _N, _N), jnp.float32),
    )(x, xt, Wrel0, Wroot0, brel0.reshape(1, -1), Wrel1, Wroot1,
      brel1.reshape(1, -1), Wrel2, Wroot2, brel2.reshape(1, -1))
    return out


# speedup vs baseline: 2.0286x; 2.0286x over previous
import jax
import jax.numpy as jnp
from jax.experimental import pallas as pl

_B, _N, _C = 256, 128, 10
_K = 10
_S = 8

_DN = (((1,), (0,)), ((), ()))


def _one_set(xb, xt, weights):
    (wr0, wo0, b0, wr1, wo1, b1, wr2, wo2, b2) = weights

    acc = jnp.zeros((_N, _N), jnp.float32)
    for c in range(_C):
        d = xb[:, c:c + 1] - xt[c:c + 1, :]
        acc = acc + d * d
    dist = jnp.sqrt(acc)

    iota = jax.lax.broadcasted_iota(jnp.int32, (_N, _N), 0).astype(jnp.float32)
    adj_t = jnp.zeros((_N, _N), jnp.float32)
    dcur = dist
    for t in range(_K + 1):
        v, ix = dcur, iota
        rows = _N
        while rows > 8:
            rows //= 2
            av, bv = v[:rows], v[rows:]
            ai, bi = ix[:rows], ix[rows:]
            take_b = bv < av
            v = jnp.minimum(av, bv)
            ix = jnp.where(take_b, bi, ai)
        for sh in (4, 2, 1):
            bv = jnp.roll(v, -sh, axis=0)
            bi = jnp.roll(ix, -sh, axis=0)
            take_b = (bv < v) | ((bv == v) & (bi < ix))
            v = jnp.where(take_b, bv, v)
            ix = jnp.where(take_b, bi, ix)
        onehot = iota == ix[0:1, :]
        if t > 0:
            adj_t = adj_t + jnp.where(onehot, 1.0, 0.0)
        dcur = jnp.where(onehot, jnp.inf, dcur)

    def conv(h, wr, wo, b):
        agg = jax.lax.dot_general(adj_t, h, _DN, preferred_element_type=jnp.float32)
        return (jax.lax.dot_general(agg, wr, _DN, preferred_element_type=jnp.float32)
                + b
                + jax.lax.dot_general(h, wo, _DN, preferred_element_type=jnp.float32))

    h = conv(xb, wr0, wo0, b0)
    h = jnp.maximum(h, 0.0)
    h = conv(h, wr1, wo1, b1)
    h = jnp.maximum(h, 0.0)
    h = conv(h, wr2, wo2, b2)

    return jax.lax.dot_general(h, h, (((1,), (1,)), ((), ())),
                               preferred_element_type=jnp.float32)


def _body(x_ref, xt_ref, wr0_ref, wo0_ref, b0_ref, wr1_ref, wo1_ref, b1_ref,
          wr2_ref, wo2_ref, b2_ref, out_ref):
    weights = (wr0_ref[...], wo0_ref[...], b0_ref[...],
               wr1_ref[...], wo1_ref[...], b1_ref[...],
               wr2_ref[...], wo2_ref[...], b2_ref[...])
    for s in range(_S):
        out_ref[s, 0] = _one_set(x_ref[s], xt_ref[s], weights)


def kernel(x, Wrel0, Wroot0, brel0, Wrel1, Wroot1, brel1, Wrel2, Wroot2, brel2):
    xt = jnp.transpose(x, (0, 2, 1))
    full = lambda s: pl.BlockSpec(s, lambda i: (0,) * len(s))
    grid_spec = pl.GridSpec(
        grid=(_B // _S,),
        in_specs=[
            pl.BlockSpec((_S, _N, _C), lambda i: (i, 0, 0)),
            pl.BlockSpec((_S, _C, _N), lambda i: (i, 0, 0)),
            full((10, 64)), full((10, 64)), full((1, 64)),
            full((64, 64)), full((64, 64)), full((1, 64)),
            full((64, 32)), full((64, 32)), full((1, 32)),
        ],
        out_specs=pl.BlockSpec((_S, 1, _N, _N), lambda i: (i, 0, 0, 0)),
    )
    out = pl.pallas_call(
        _body,
        grid_spec=grid_spec,
        out_shape=jax.ShapeDtypeStruct((_B, 1, _N, _N), jnp.float32),
    )(x, xt, Wrel0, Wroot0, brel0.reshape(1, -1), Wrel1, Wroot1,
      brel1.reshape(1, -1), Wrel2, Wroot2, brel2.reshape(1, -1))
    return out
